# baseline (speedup 1.0000x reference)
import jax
import jax.numpy as jnp
from jax import lax
from jax.experimental import pallas as pl

B, N, VOCAB, EMB = 1024, 50, 1000, 64
B_BLK = 16


def _argmax_gather_block(x_ref, w_ref, out_ref):
    xb = x_ref[...]
    mx = jnp.max(xb, axis=2, keepdims=True)
    iota = lax.broadcasted_iota(jnp.int32, xb.shape, 2)
    idx = jnp.min(jnp.where(xb == mx, iota, VOCAB), axis=2)
    oh = (
        lax.broadcasted_iota(jnp.int32, (B_BLK * N, VOCAB), 1)
        == idx.reshape(B_BLK * N, 1)
    ).astype(jnp.float32)
    res = jax.lax.dot(oh, w_ref[...], preferred_element_type=jnp.float32)
    out_ref[...] = res.reshape(B_BLK, N, EMB)


def kernel(x, W):
    return pl.pallas_call(
        _argmax_gather_block,
        grid=(B // B_BLK,),
        in_specs=[
            pl.BlockSpec((B_BLK, N, VOCAB), lambda i: (i, 0, 0)),
            pl.BlockSpec((VOCAB, EMB), lambda i: (0, 0)),
        ],
        out_specs=pl.BlockSpec((B_BLK, N, EMB), lambda i: (i, 0, 0)),
        out_shape=jax.ShapeDtypeStruct((B, N, EMB), jnp.float32),
    )(x, W)
